# baseline (speedup 1.0000x reference)
import jax
import jax.numpy as jnp
from jax import lax
from jax.experimental import pallas as pl
from jax.experimental.pallas import tpu as pltpu
from jax.experimental.pallas import tpu_sc as plsc

NC = 2
NS = 16
NW = NC * NS

B = 16384
L = 50
D = 64
ROWS_W = B // NW
NBUF = 4


def _body(table_hbm, x_hbm, out_hbm, idx_v, rows_v, gsems, osems):
    wid = lax.axis_index("s") * NC + lax.axis_index("c")
    base = wid * ROWS_W
    pltpu.sync_copy(x_hbm.at[pl.ds(base, ROWS_W)], idx_v)

    for b in range(NBUF):
        pltpu.async_copy(table_hbm.at[idx_v.at[b]], rows_v.at[b], gsems.at[b])

    @pl.loop(0, ROWS_W, step=NBUF)
    def _(j):
        for b in range(NBUF):
            r = j + b
            pltpu.make_async_copy(
                table_hbm.at[idx_v.at[r]], rows_v.at[b], gsems.at[b]
            ).wait()
            pltpu.async_copy(rows_v.at[b], out_hbm.at[base + r], osems.at[b])

            @pl.when(r + NBUF < ROWS_W)
            def _():
                pltpu.make_async_copy(
                    rows_v.at[b], out_hbm.at[base + r], osems.at[b]
                ).wait()
                pltpu.async_copy(
                    table_hbm.at[idx_v.at[r + NBUF]],
                    rows_v.at[b],
                    gsems.at[b],
                )

    for b in range(NBUF):
        pltpu.make_async_copy(
            rows_v.at[b], out_hbm.at[base + ROWS_W - NBUF + b], osems.at[b]
        ).wait()


@jax.jit
def _gather(table, x):
    mesh = plsc.VectorSubcoreMesh(
        core_axis_name="c", subcore_axis_name="s", num_cores=NC, num_subcores=NS
    )
    return pl.kernel(
        _body,
        out_type=jax.ShapeDtypeStruct((B, L, D), jnp.float32),
        mesh=mesh,
        scratch_types=[
            pltpu.VMEM((ROWS_W, L), jnp.int32),
            pltpu.VMEM((NBUF, L, D), jnp.float32),
            pltpu.SemaphoreType.DMA((NBUF,)),
            pltpu.SemaphoreType.DMA((NBUF,)),
        ],
        compiler_params=pltpu.CompilerParams(use_tc_tiling_on_sc=False),
    )(table, x)


def kernel(x, table):
    return _gather(table, x.astype(jnp.int32))
